# baseline (speedup 1.0000x reference)
import jax
import jax.numpy as jnp
from jax import lax
from jax.experimental import pallas as pl
from jax.experimental.pallas import tpu as pltpu
from jax.experimental.pallas import tpu_sc as plsc

_BATCH = 4096
_FIELDS = 26
_VOCAB = 100000
_DIM = 64
_L = 16
_NB = _BATCH // _L
_QSPAN = 24576
_CH = 4992
_MAIN = ((0, 4992), (4992, 4992), (9984, 4992), (14976, 4992), (19968, 4608))
_TAIL_LO = 4 * _QSPAN
_TAIL_MAIN = 1664
_EDGE_LO = _TAIL_LO + _TAIL_MAIN
_EDGE = _VOCAB - _EDGE_LO


def _body(xt_hbm, tt_hbm, out_hbm, part_hbm, xbuf, vbufa, vbufb, ebuf, acc,
          sema, semb, semx, seme):
    c = lax.axis_index("c")
    s = lax.axis_index("s")
    db = c * 4 + (s % 4)
    q = s // 4
    d0 = db * 8
    qlo = q * _QSPAN
    zero = jnp.zeros((_L,), jnp.float32)
    iota = lax.iota(jnp.int32, _L)

    @plsc.parallel_loop(0, _NB, unroll=4)
    def _zero_acc(i):
        for dl in range(8):
            acc[dl, pl.ds(i * _L, _L)] = zero

    def chunk_src(f, ci):
        row0 = pl.multiple_of(f * _DIM + d0, 8)
        if ci < 5:
            lo_c, ext = _MAIN[ci]
            lo = pl.multiple_of(qlo + lo_c, 128)
        else:
            lo, ext = _TAIL_LO, _TAIL_MAIN
        return tt_hbm.at[pl.ds(row0, 8), pl.ds(lo, ext)], ext

    def issue(f, ci, buf, sem):
        src, ext = chunk_src(f, ci)
        dst = buf if ext == _CH else buf.at[:, pl.ds(0, ext)]
        pltpu.async_copy(src, dst, sem)

    def wait(buf, sem, ext):
        pltpu.make_async_copy(
            tt_hbm.at[pl.ds(0, 8), pl.ds(0, ext)],
            buf if ext == _CH else buf.at[:, pl.ds(0, ext)],
            sem).wait()

    def gather_pass(buf, xbase, lo, ch, b_lo, b_n):
        @plsc.parallel_loop(0, b_n, unroll=8)
        def per_bvec(i):
            b = b_lo + i
            idx = xbuf[pl.ds(xbase + b * _L, _L)]
            idxc = idx - lo
            inb = (idxc >= 0) & (idxc < ch)
            idxg = lax.min(lax.max(idxc, 0), ch - 1)
            pos = iota + b * _L
            for dl in range(8):
                row = jnp.full((_L,), dl, jnp.int32)
                g = plsc.load_gather(buf, [row, idxg])
                plsc.addupdate_scatter(acc, [row, pos], g, mask=inb)

    def issue_xbuf(f):
        base = pl.multiple_of(lax.rem(f, 2) * _BATCH, _BATCH)
        pltpu.async_copy(
            xt_hbm.at[pl.ds(pl.multiple_of(f * _BATCH, _BATCH), _BATCH)],
            xbuf.at[pl.ds(base, _BATCH)], semx)

    issue_xbuf(0)
    issue(0, 0, vbufa, sema)
    issue(0, 1, vbufb, semb)

    def per_field(f, carry):
        pltpu.make_async_copy(
            xt_hbm.at[pl.ds(0, _BATCH)], xbuf.at[pl.ds(0, _BATCH)],
            semx).wait()

        @pl.when(f < _FIELDS - 1)
        def _nextx():
            issue_xbuf(f + 1)

        xbase = pl.multiple_of(lax.rem(f, 2) * _BATCH, _BATCH)
        row0 = pl.multiple_of(f * _DIM + d0, 8)
        pltpu.async_copy(tt_hbm.at[pl.ds(row0, 8), pl.ds(_EDGE_LO, _EDGE)],
                         ebuf, seme)
        for ci in range(6):
            buf, sem = (vbufa, sema) if ci % 2 == 0 else (vbufb, semb)
            if ci < 5:
                lo_c, ext = _MAIN[ci]
                wait(buf, sem, ext)
                gather_pass(buf, xbase, qlo + lo_c, ext, 0, _NB)
            else:
                wait(buf, sem, _TAIL_MAIN)
                gather_pass(buf, xbase, _TAIL_LO, _TAIL_MAIN,
                            q * (_NB // 4), _NB // 4)
            if ci < 4:
                issue(f, ci + 2, buf, sem)
            else:

                @pl.when(f < _FIELDS - 1)
                def _crossfield(ci=ci):
                    issue(f + 1, ci - 4, buf, sem)

        pltpu.make_async_copy(
            tt_hbm.at[pl.ds(0, 8), pl.ds(_EDGE_LO, _EDGE)], ebuf,
            seme).wait()
        gather_pass(ebuf, xbase, _EDGE_LO, _EDGE, q * (_NB // 4), _NB // 4)
        return carry

    lax.fori_loop(0, _FIELDS, per_field, 0)

    @pl.when(s >= 4)
    def _publish():
        pltpu.sync_copy(acc, part_hbm.at[c * 16 + s])

    plsc.subcore_barrier()

    @pl.when(s < 4)
    def _reduce():
        for qq in range(1, 4):
            pltpu.sync_copy(part_hbm.at[c * 16 + qq * 4 + s],
                            vbufa.at[:, pl.ds(0, _BATCH)])

            @plsc.parallel_loop(0, _NB, unroll=4)
            def add_vec(j):
                for dl in range(8):
                    sl = pl.ds(j * _L, _L)
                    plsc.addupdate(acc.at[dl, sl], vbufa[dl, sl])

        dout = pl.multiple_of((c * 4 + s) * 8, 8)
        pltpu.sync_copy(acc, out_hbm.at[pl.ds(dout, 8)])


def kernel(x, tables):
    tt = tables.transpose(0, 2, 1).reshape(_FIELDS * _DIM, _VOCAB)
    xt = x.T.reshape(_FIELDS * _BATCH)
    mesh = plsc.VectorSubcoreMesh(core_axis_name="c", subcore_axis_name="s")
    k = pl.kernel(
        _body,
        out_type=(
            jax.ShapeDtypeStruct((_DIM, _BATCH), jnp.float32),
            jax.ShapeDtypeStruct((32, 8, _BATCH), jnp.float32),
        ),
        mesh=mesh,
        compiler_params=pltpu.CompilerParams(needs_layout_passes=False),
        scratch_types=[
            pltpu.VMEM((2 * _BATCH,), jnp.int32),
            pltpu.VMEM((8, _CH), jnp.float32),
            pltpu.VMEM((8, _CH), jnp.float32),
            pltpu.VMEM((8, _EDGE), jnp.float32),
            pltpu.VMEM((8, _BATCH), jnp.float32),
            pltpu.SemaphoreType.DMA,
            pltpu.SemaphoreType.DMA,
            pltpu.SemaphoreType.DMA,
            pltpu.SemaphoreType.DMA,
        ],
    )
    out_t, _ = k(xt, tt)
    return out_t.T


# speedup vs baseline: 1.9513x; 1.9513x over previous
import jax
import jax.numpy as jnp
from jax import lax
from jax.experimental import pallas as pl
from jax.experimental.pallas import tpu as pltpu
from jax.experimental.pallas import tpu_sc as plsc

_BATCH = 4096
_FIELDS = 26
_VOCAB = 100000
_DIM = 64
_L = 16
_NB = _BATCH // _L
_QSPAN = 24576
_CH = 4992
_MAIN = ((0, 4992), (4992, 4992), (9984, 4992), (14976, 4992), (19968, 4608))
_TAIL_LO = 4 * _QSPAN
_TAIL_MAIN = 1664
_EDGE_LO = _TAIL_LO + _TAIL_MAIN
_EDGE = _VOCAB - _EDGE_LO


def _body(xt_hbm, tt_hbm, out_hbm, part_hbm, xbuf, vbufa, vbufb, ebuf, acc,
          sema, semb, semx, seme):
    c = lax.axis_index("c")
    s = lax.axis_index("s")
    db = c * 4 + (s % 4)
    q = s // 4
    d0 = db * 8
    qlo = q * _QSPAN
    zero = jnp.zeros((_L,), jnp.float32)
    iota = lax.iota(jnp.int32, _L)

    @plsc.parallel_loop(0, _NB, unroll=4)
    def _zero_acc(i):
        for dl in range(8):
            acc[dl, pl.ds(i * _L, _L)] = zero

    def chunk_src(f, ci):
        row0 = pl.multiple_of(f * _DIM + d0, 8)
        if ci < 5:
            lo_c, ext = _MAIN[ci]
            lo = pl.multiple_of(qlo + lo_c, 128)
        else:
            lo, ext = _TAIL_LO, _TAIL_MAIN
        return tt_hbm.at[pl.ds(row0, 8), pl.ds(lo, ext)], ext

    def issue(f, ci, buf, sem):
        src, ext = chunk_src(f, ci)
        dst = buf if ext == _CH else buf.at[:, pl.ds(0, ext)]
        pltpu.async_copy(src, dst, sem)

    def wait(buf, sem, ext):
        pltpu.make_async_copy(
            tt_hbm.at[pl.ds(0, 8), pl.ds(0, ext)],
            buf if ext == _CH else buf.at[:, pl.ds(0, ext)],
            sem).wait()

    def gather_pass(buf, xbase, lo, ch, b_lo, b_n):
        @plsc.parallel_loop(0, b_n, unroll=4)
        def per_bvec(i):
            b = b_lo + i
            idx = xbuf[pl.ds(xbase + b * _L, _L)]
            idxc = idx - lo
            inb = (idxc >= 0) & (idxc < ch)
            idxg = lax.min(lax.max(idxc, 0), ch - 1)
            pos = iota + b * _L
            for dl in range(8):
                row = jnp.full((_L,), dl, jnp.int32)
                g = plsc.load_gather(buf, [row, idxg])
                plsc.addupdate_scatter(acc, [row, pos], g, mask=inb)

    def issue_xbuf(f):
        base = pl.multiple_of(lax.rem(f, 2) * _BATCH, _BATCH)
        pltpu.async_copy(
            xt_hbm.at[pl.ds(pl.multiple_of(f * _BATCH, _BATCH), _BATCH)],
            xbuf.at[pl.ds(base, _BATCH)], semx)

    issue_xbuf(0)
    issue(0, 0, vbufa, sema)
    issue(0, 1, vbufb, semb)

    def per_field(f, carry):
        pltpu.make_async_copy(
            xt_hbm.at[pl.ds(0, _BATCH)], xbuf.at[pl.ds(0, _BATCH)],
            semx).wait()

        @pl.when(f < _FIELDS - 1)
        def _nextx():
            issue_xbuf(f + 1)

        xbase = pl.multiple_of(lax.rem(f, 2) * _BATCH, _BATCH)
        row0 = pl.multiple_of(f * _DIM + d0, 8)
        pltpu.async_copy(tt_hbm.at[pl.ds(row0, 8), pl.ds(_EDGE_LO, _EDGE)],
                         ebuf, seme)
        for ci in range(6):
            buf, sem = (vbufa, sema) if ci % 2 == 0 else (vbufb, semb)
            if ci < 5:
                lo_c, ext = _MAIN[ci]
                wait(buf, sem, ext)
                gather_pass(buf, xbase, qlo + lo_c, ext, 0, _NB)
            else:
                wait(buf, sem, _TAIL_MAIN)
                gather_pass(buf, xbase, _TAIL_LO, _TAIL_MAIN,
                            q * (_NB // 4), _NB // 4)
            if ci < 4:
                issue(f, ci + 2, buf, sem)
            else:

                @pl.when(f < _FIELDS - 1)
                def _crossfield(ci=ci):
                    issue(f + 1, ci - 4, buf, sem)

        pltpu.make_async_copy(
            tt_hbm.at[pl.ds(0, 8), pl.ds(_EDGE_LO, _EDGE)], ebuf,
            seme).wait()
        gather_pass(ebuf, xbase, _EDGE_LO, _EDGE, q * (_NB // 4), _NB // 4)
        return carry

    lax.fori_loop(0, _FIELDS, per_field, 0)

    @pl.when(s >= 4)
    def _publish():
        pltpu.sync_copy(acc, part_hbm.at[c * 16 + s])

    plsc.subcore_barrier()

    @pl.when(s < 4)
    def _reduce():
        for qq in range(1, 4):
            pltpu.sync_copy(part_hbm.at[c * 16 + qq * 4 + s],
                            vbufa.at[:, pl.ds(0, _BATCH)])

            @plsc.parallel_loop(0, _NB, unroll=4)
            def add_vec(j):
                for dl in range(8):
                    sl = pl.ds(j * _L, _L)
                    plsc.addupdate(acc.at[dl, sl], vbufa[dl, sl])

        dout = pl.multiple_of((c * 4 + s) * 8, 8)
        pltpu.sync_copy(acc, out_hbm.at[pl.ds(dout, 8)])


def kernel(x, tables):
    tt = tables.transpose(0, 2, 1).reshape(_FIELDS * _DIM, _VOCAB)
    xt = x.T.reshape(_FIELDS * _BATCH)
    mesh = plsc.VectorSubcoreMesh(core_axis_name="c", subcore_axis_name="s")
    k = pl.kernel(
        _body,
        out_type=(
            jax.ShapeDtypeStruct((_DIM, _BATCH), jnp.float32),
            jax.ShapeDtypeStruct((32, 8, _BATCH), jnp.float32),
        ),
        mesh=mesh,
        compiler_params=pltpu.CompilerParams(needs_layout_passes=False),
        scratch_types=[
            pltpu.VMEM((2 * _BATCH,), jnp.int32),
            pltpu.VMEM((8, _CH), jnp.float32),
            pltpu.VMEM((8, _CH), jnp.float32),
            pltpu.VMEM((8, _EDGE), jnp.float32),
            pltpu.VMEM((8, _BATCH), jnp.float32),
            pltpu.SemaphoreType.DMA,
            pltpu.SemaphoreType.DMA,
            pltpu.SemaphoreType.DMA,
            pltpu.SemaphoreType.DMA,
        ],
    )
    out_t, _ = k(xt, tt)
    return out_t.T
